# Initial kernel scaffold; baseline (speedup 1.0000x reference)
import functools

import jax
import jax.numpy as jnp
from jax import lax
from jax.experimental import pallas as pl
from jax.experimental.pallas import tpu as pltpu
from jax.experimental.pallas import tpu_sc as plsc

N = 10000
NP = 10240
D = 128
E = 320000
EB = 128
NCHUNK = E // EB
NC, NS = 2, 16
ROWS_PER_CORE = NCHUNK // NC
JMAX = (ROWS_PER_CORE + NS - 1) // NS
RPS = NP // NS
WCH = RPS // EB

_mesh = plsc.VectorSubcoreMesh(
    core_axis_name="c", subcore_axis_name="s", num_cores=NC, num_subcores=NS)

_f32 = jnp.float32


@functools.partial(
    pl.kernel,
    out_type=(jax.ShapeDtypeStruct((NP,), _f32),
              jax.ShapeDtypeStruct((NP,), _f32)),
    mesh=_mesh,
    scratch_types=[
        pltpu.VMEM((EB,), jnp.int32),
        pltpu.VMEM((EB,), _f32),
        pltpu.VMEM((RPS,), _f32),
        pltpu.VMEM_SHARED((NP,), _f32),
    ],
)
def _deg_kernel(dst_hbm, out_a, out_b, dst_v, ones_v, stage_v, acc_sh):
    c = lax.axis_index("c")
    s = lax.axis_index("s")

    def _fill(t, _):
        ones_v[pl.ds(t * 16, 16)] = jnp.full((16,), 1.0, _f32)
        return 0
    lax.fori_loop(0, EB // 16, _fill, 0)

    def _zero(t, _):
        stage_v[pl.ds(t * 16, 16)] = jnp.zeros((16,), _f32)
        return 0
    lax.fori_loop(0, RPS // 16, _zero, 0)

    pltpu.sync_copy(stage_v, acc_sh.at[pl.ds(s * RPS, RPS)])
    plsc.subcore_barrier()

    def _edges(j, _):
        r = s + NS * j

        @pl.when(r < ROWS_PER_CORE)
        def _():
            row = c * ROWS_PER_CORE + r
            pltpu.sync_copy(dst_hbm.at[pl.ds(row * EB, EB)], dst_v)
            pltpu.sync_copy(ones_v, acc_sh.at[dst_v], add=True)
        return 0
    lax.fori_loop(0, JMAX, _edges, 0)
    plsc.subcore_barrier()

    pltpu.sync_copy(acc_sh.at[pl.ds(s * RPS, RPS)], stage_v)

    @pl.when(c == 0)
    def _():
        pltpu.sync_copy(stage_v, out_a.at[pl.ds(s * RPS, RPS)])

    @pl.when(c == 1)
    def _():
        pltpu.sync_copy(stage_v, out_b.at[pl.ds(s * RPS, RPS)])


@functools.partial(
    pl.kernel,
    out_type=(jax.ShapeDtypeStruct((NP, D), _f32),
              jax.ShapeDtypeStruct((NP, D), _f32)),
    mesh=_mesh,
    scratch_types=[
        pltpu.VMEM((EB,), jnp.int32),
        pltpu.VMEM((EB,), jnp.int32),
        pltpu.VMEM((EB, D), _f32),
        pltpu.VMEM_SHARED((NP, D), _f32),
        pltpu.SemaphoreType.DMA,
    ],
)
def _edge_kernel(u_hbm, src_hbm, dst_hbm, out_a, out_b,
                 src_v, dst_v, rows_v, acc_sh, sem):
    c = lax.axis_index("c")
    s = lax.axis_index("s")

    def _zero(t, _):
        rows_v[t // 8, pl.ds((t % 8) * 16, 16)] = jnp.zeros((16,), _f32)
        return 0
    lax.fori_loop(0, EB * D // 16, _zero, 0)

    def _zacc(k, _):
        pltpu.sync_copy(rows_v, acc_sh.at[pl.ds(s * RPS + k * EB, EB)])
        return 0
    lax.fori_loop(0, WCH, _zacc, 0)
    plsc.subcore_barrier()

    def _edges(j, _):
        r = s + NS * j

        @pl.when(r < ROWS_PER_CORE)
        def _():
            row = c * ROWS_PER_CORE + r
            pltpu.sync_copy(src_hbm.at[pl.ds(row * EB, EB)], src_v)
            pltpu.sync_copy(dst_hbm.at[pl.ds(row * EB, EB)], dst_v)
            pltpu.async_copy(u_hbm.at[src_v], rows_v, sem).wait()
            pltpu.sync_copy(rows_v, acc_sh.at[dst_v], add=True)
        return 0
    lax.fori_loop(0, JMAX, _edges, 0)
    plsc.subcore_barrier()

    def _wout(k, _):
        pltpu.sync_copy(acc_sh.at[pl.ds(s * RPS + k * EB, EB)], rows_v)

        @pl.when(c == 0)
        def _():
            pltpu.sync_copy(rows_v, out_a.at[pl.ds(s * RPS + k * EB, EB)])

        @pl.when(c == 1)
        def _():
            pltpu.sync_copy(rows_v, out_b.at[pl.ds(s * RPS + k * EB, EB)])
        return 0
    lax.fori_loop(0, WCH, _wout, 0)


_BLK = 1024
_GRID = NP // _BLK


def _dot_t(a, w):
    return lax.dot_general(a, w, (((1,), (1,)), ((), ())),
                           preferred_element_type=_f32)


def _tc1_body(x_ref, da_ref, db_ref, w_ref, u_ref, dinv_ref):
    dinv = lax.rsqrt(da_ref[...] + db_ref[...] + 1.0)
    u_ref[...] = dinv * _dot_t(x_ref[...], w_ref[...])
    dinv_ref[...] = dinv


def _tc1(xp, dega, degb, w1):
    return pl.pallas_call(
        _tc1_body,
        grid=(_GRID,),
        in_specs=[
            pl.BlockSpec((_BLK, D), lambda i: (i, 0)),
            pl.BlockSpec((_BLK, 1), lambda i: (i, 0)),
            pl.BlockSpec((_BLK, 1), lambda i: (i, 0)),
            pl.BlockSpec((D, D), lambda i: (0, 0)),
        ],
        out_specs=[
            pl.BlockSpec((_BLK, D), lambda i: (i, 0)),
            pl.BlockSpec((_BLK, 1), lambda i: (i, 0)),
        ],
        out_shape=[
            jax.ShapeDtypeStruct((NP, D), _f32),
            jax.ShapeDtypeStruct((NP, 1), _f32),
        ],
    )(xp, dega, degb, w1)


def _tc_mid_body(za_ref, zb_ref, u_ref, dinv_ref, b_ref, w_ref, un_ref):
    dinv = dinv_ref[...]
    h = dinv * (za_ref[...] + zb_ref[...] + u_ref[...]) + b_ref[...]
    h = jnp.maximum(h, 0.0)
    un_ref[...] = dinv * _dot_t(h, w_ref[...])


def _tc_mid(za, zb, u, dinv, b, w):
    return pl.pallas_call(
        _tc_mid_body,
        grid=(_GRID,),
        in_specs=[
            pl.BlockSpec((_BLK, D), lambda i: (i, 0)),
            pl.BlockSpec((_BLK, D), lambda i: (i, 0)),
            pl.BlockSpec((_BLK, D), lambda i: (i, 0)),
            pl.BlockSpec((_BLK, 1), lambda i: (i, 0)),
            pl.BlockSpec((1, D), lambda i: (0, 0)),
            pl.BlockSpec((D, D), lambda i: (0, 0)),
        ],
        out_specs=pl.BlockSpec((_BLK, D), lambda i: (i, 0)),
        out_shape=jax.ShapeDtypeStruct((NP, D), _f32),
    )(za, zb, u, dinv, b, w)


def _sigmoid(t):
    return 1.0 / (1.0 + jnp.exp(-t))


def _tc_final_body(za_ref, zb_ref, u_ref, dinv_ref, b_ref,
                   ws1_ref, bs1_ref, ws2_ref, bs2_ref,
                   wi1_ref, bi1_ref, wi2_ref, bi2_ref,
                   score_ref, issues_ref, acc_ref):
    i = pl.program_id(0)
    h = dinv_ref[...] * (za_ref[...] + zb_ref[...] + u_ref[...]) + b_ref[...]
    h = jnp.maximum(h, 0.0)
    rows = lax.broadcasted_iota(jnp.int32, (_BLK, 1), 0) + i * _BLK
    h = h * (rows < N).astype(_f32)

    @pl.when(i == 0)
    def _():
        acc_ref[...] = jnp.zeros((1, D), _f32)

    acc_ref[...] += jnp.sum(h, axis=0, keepdims=True)

    @pl.when(i == _GRID - 1)
    def _():
        g = acc_ref[...] * (1.0 / N)
        t1 = jnp.maximum(_dot_t(g, ws1_ref[...]) + bs1_ref[...], 0.0)
        score_ref[...] = _sigmoid(_dot_t(t1, ws2_ref[...]) + bs2_ref[...])
        t2 = jnp.maximum(_dot_t(g, wi1_ref[...]) + bi1_ref[...], 0.0)
        issues_ref[...] = _sigmoid(_dot_t(t2, wi2_ref[...]) + bi2_ref[...])


def _tc_final(za, zb, u, dinv, b, ws1, bs1, ws2, bs2, wi1, bi1, wi2, bi2):
    full = lambda shp: pl.BlockSpec(shp, lambda i: (0, 0))
    return pl.pallas_call(
        _tc_final_body,
        grid=(_GRID,),
        in_specs=[
            pl.BlockSpec((_BLK, D), lambda i: (i, 0)),
            pl.BlockSpec((_BLK, D), lambda i: (i, 0)),
            pl.BlockSpec((_BLK, D), lambda i: (i, 0)),
            pl.BlockSpec((_BLK, 1), lambda i: (i, 0)),
            full((1, D)),
            full((D // 2, D)), full((1, D // 2)),
            full((1, D // 2)), full((1, 1)),
            full((D // 2, D)), full((1, D // 2)),
            full((10, D // 2)), full((1, 10)),
        ],
        out_specs=[full((1, 1)), full((1, 10))],
        out_shape=[
            jax.ShapeDtypeStruct((1, 1), _f32),
            jax.ShapeDtypeStruct((1, 10), _f32),
        ],
        scratch_shapes=[pltpu.VMEM((1, D), _f32)],
    )(za, zb, u, dinv, b, ws1, bs1, ws2, bs2, wi1, bi1, wi2, bi2)


def kernel(x, edge_index, W1, b1, W2, b2, W3, b3,
           Ws1, bs1, Ws2, bs2, Wi1, bi1, Wi2, bi2):
    xp = jnp.pad(x, ((0, NP - N), (0, 0)))
    src = edge_index[0]
    dst = edge_index[1]
    b1r = b1.reshape(1, D)
    b2r = b2.reshape(1, D)
    b3r = b3.reshape(1, D)
    bs1r = bs1.reshape(1, D // 2)
    bs2r = bs2.reshape(1, 1)
    bi1r = bi1.reshape(1, D // 2)
    bi2r = bi2.reshape(1, 10)

    dega, degb = _deg_kernel(dst)
    u1, dinv = _tc1(xp, dega.reshape(NP, 1), degb.reshape(NP, 1), W1)
    za, zb = _edge_kernel(u1, src, dst)
    u2 = _tc_mid(za, zb, u1, dinv, b1r, W2)
    za, zb = _edge_kernel(u2, src, dst)
    u3 = _tc_mid(za, zb, u2, dinv, b2r, W3)
    za, zb = _edge_kernel(u3, src, dst)
    score, issues = _tc_final(za, zb, u3, dinv, b3r,
                              Ws1, bs1r, Ws2, bs2r, Wi1, bi1r, Wi2, bi2r)
    return (score, issues)


# speedup vs baseline: 13.9544x; 13.9544x over previous
import functools

import jax
import jax.numpy as jnp
from jax import lax
from jax.experimental import pallas as pl
from jax.experimental.pallas import tpu as pltpu
from jax.experimental.pallas import tpu_sc as plsc

N = 10000
NP = 10240
D = 128
E = 320000
EB = 128
NCHUNK = E // EB
NC, NS = 2, 16
ROWS_PER_CORE = NCHUNK // NC
JMAX = (ROWS_PER_CORE + NS - 1) // NS
RPS = NP // NS
WCH = RPS // EB

_mesh = plsc.VectorSubcoreMesh(
    core_axis_name="c", subcore_axis_name="s", num_cores=NC, num_subcores=NS)

_f32 = jnp.float32


@functools.partial(
    pl.kernel,
    out_type=(jax.ShapeDtypeStruct((NP,), _f32),
              jax.ShapeDtypeStruct((NP,), _f32)),
    mesh=_mesh,
    scratch_types=[
        pltpu.VMEM((EB,), jnp.int32),
        pltpu.VMEM((EB,), _f32),
        pltpu.VMEM((RPS,), _f32),
        pltpu.VMEM_SHARED((NP,), _f32),
    ],
)
def _deg_kernel(dst_hbm, out_a, out_b, dst_v, ones_v, stage_v, acc_sh):
    c = lax.axis_index("c")
    s = lax.axis_index("s")

    def _fill(t, _):
        ones_v[pl.ds(t * 16, 16)] = jnp.full((16,), 1.0, _f32)
        return 0
    lax.fori_loop(0, EB // 16, _fill, 0)

    def _zero(t, _):
        stage_v[pl.ds(t * 16, 16)] = jnp.zeros((16,), _f32)
        return 0
    lax.fori_loop(0, RPS // 16, _zero, 0)

    pltpu.sync_copy(stage_v, acc_sh.at[pl.ds(s * RPS, RPS)])
    plsc.subcore_barrier()

    def _edges(j, _):
        r = s + NS * j

        @pl.when(r < ROWS_PER_CORE)
        def _():
            row = c * ROWS_PER_CORE + r
            pltpu.sync_copy(dst_hbm.at[pl.ds(row * EB, EB)], dst_v)
            pltpu.sync_copy(ones_v, acc_sh.at[dst_v], add=True)
        return 0
    lax.fori_loop(0, JMAX, _edges, 0)
    plsc.subcore_barrier()

    pltpu.sync_copy(acc_sh.at[pl.ds(s * RPS, RPS)], stage_v)

    @pl.when(c == 0)
    def _():
        pltpu.sync_copy(stage_v, out_a.at[pl.ds(s * RPS, RPS)])

    @pl.when(c == 1)
    def _():
        pltpu.sync_copy(stage_v, out_b.at[pl.ds(s * RPS, RPS)])


@functools.partial(
    pl.kernel,
    out_type=(jax.ShapeDtypeStruct((NP, D), _f32),
              jax.ShapeDtypeStruct((NP, D), _f32)),
    mesh=_mesh,
    scratch_types=[
        pltpu.VMEM((EB,), jnp.int32),
        pltpu.VMEM((EB,), jnp.int32),
        pltpu.VMEM((EB, D), _f32),
        pltpu.VMEM_SHARED((NP, D), _f32),
        pltpu.SemaphoreType.DMA,
    ],
)
def _edge_kernel(u_hbm, src_hbm, dst_hbm, out_a, out_b,
                 src_v, dst_v, rows_v, acc_sh, sem):
    c = lax.axis_index("c")
    s = lax.axis_index("s")

    def _zero(t, _):
        rows_v[t // 8, pl.ds((t % 8) * 16, 16)] = jnp.zeros((16,), _f32)
        return 0
    lax.fori_loop(0, EB * D // 16, _zero, 0)

    def _zacc(k, _):
        pltpu.sync_copy(rows_v, acc_sh.at[pl.ds(s * RPS + k * EB, EB)])
        return 0
    lax.fori_loop(0, WCH, _zacc, 0)
    plsc.subcore_barrier()

    def _edges(j, _):
        r = s + NS * j

        @pl.when(r < ROWS_PER_CORE)
        def _():
            row = c * ROWS_PER_CORE + r
            pltpu.sync_copy(src_hbm.at[pl.ds(row * EB, EB)], src_v)
            pltpu.sync_copy(dst_hbm.at[pl.ds(row * EB, EB)], dst_v)
            pltpu.async_copy(u_hbm.at[src_v], rows_v, sem).wait()
            pltpu.sync_copy(rows_v, acc_sh.at[dst_v], add=True)
        return 0
    lax.fori_loop(0, JMAX, _edges, 0)
    plsc.subcore_barrier()

    def _wout(k, _):
        pltpu.sync_copy(acc_sh.at[pl.ds(s * RPS + k * EB, EB)], rows_v)

        @pl.when(c == 0)
        def _():
            pltpu.sync_copy(rows_v, out_a.at[pl.ds(s * RPS + k * EB, EB)])

        @pl.when(c == 1)
        def _():
            pltpu.sync_copy(rows_v, out_b.at[pl.ds(s * RPS + k * EB, EB)])
        return 0
    lax.fori_loop(0, WCH, _wout, 0)


_BLK = 1024
_GRID = NP // _BLK


def _dot_t(a, w):
    return lax.dot_general(a, w, (((1,), (1,)), ((), ())),
                           preferred_element_type=_f32)


def _tc1_body(x_ref, da_ref, db_ref, w_ref, u_ref, dinv_ref):
    dinv = lax.rsqrt(da_ref[...] + db_ref[...] + 1.0)
    u_ref[...] = dinv * _dot_t(x_ref[...], w_ref[...])
    dinv_ref[...] = dinv


def _tc1(xp, dega, degb, w1):
    return pl.pallas_call(
        _tc1_body,
        grid=(_GRID,),
        in_specs=[
            pl.BlockSpec((_BLK, D), lambda i: (i, 0)),
            pl.BlockSpec((_BLK, 1), lambda i: (i, 0)),
            pl.BlockSpec((_BLK, 1), lambda i: (i, 0)),
            pl.BlockSpec((D, D), lambda i: (0, 0)),
        ],
        out_specs=[
            pl.BlockSpec((_BLK, D), lambda i: (i, 0)),
            pl.BlockSpec((_BLK, 1), lambda i: (i, 0)),
        ],
        out_shape=[
            jax.ShapeDtypeStruct((NP, D), _f32),
            jax.ShapeDtypeStruct((NP, 1), _f32),
        ],
    )(xp, dega, degb, w1)


def _tc_mid_body(za_ref, zb_ref, u_ref, dinv_ref, b_ref, w_ref, un_ref):
    dinv = dinv_ref[...]
    h = dinv * (za_ref[...] + zb_ref[...] + u_ref[...]) + b_ref[...]
    h = jnp.maximum(h, 0.0)
    un_ref[...] = dinv * _dot_t(h, w_ref[...])


def _tc_mid(za, zb, u, dinv, b, w):
    return pl.pallas_call(
        _tc_mid_body,
        grid=(_GRID,),
        in_specs=[
            pl.BlockSpec((_BLK, D), lambda i: (i, 0)),
            pl.BlockSpec((_BLK, D), lambda i: (i, 0)),
            pl.BlockSpec((_BLK, D), lambda i: (i, 0)),
            pl.BlockSpec((_BLK, 1), lambda i: (i, 0)),
            pl.BlockSpec((1, D), lambda i: (0, 0)),
            pl.BlockSpec((D, D), lambda i: (0, 0)),
        ],
        out_specs=pl.BlockSpec((_BLK, D), lambda i: (i, 0)),
        out_shape=jax.ShapeDtypeStruct((NP, D), _f32),
    )(za, zb, u, dinv, b, w)


def _sigmoid(t):
    return 1.0 / (1.0 + jnp.exp(-t))


def _tc_final_body(za_ref, zb_ref, u_ref, dinv_ref, b_ref,
                   ws1_ref, bs1_ref, ws2_ref, bs2_ref,
                   wi1_ref, bi1_ref, wi2_ref, bi2_ref,
                   score_ref, issues_ref, acc_ref):
    i = pl.program_id(0)
    h = dinv_ref[...] * (za_ref[...] + zb_ref[...] + u_ref[...]) + b_ref[...]
    h = jnp.maximum(h, 0.0)
    rows = lax.broadcasted_iota(jnp.int32, (_BLK, 1), 0) + i * _BLK
    h = h * (rows < N).astype(_f32)

    @pl.when(i == 0)
    def _():
        acc_ref[...] = jnp.zeros((1, D), _f32)

    acc_ref[...] += jnp.sum(h, axis=0, keepdims=True)

    @pl.when(i == _GRID - 1)
    def _():
        g = acc_ref[...] * (1.0 / N)
        t1 = jnp.maximum(_dot_t(g, ws1_ref[...]) + bs1_ref[...], 0.0)
        s_lin = jnp.sum(t1 * ws2_ref[...], axis=1, keepdims=True)
        score_ref[...] = _sigmoid(s_lin + bs2_ref[...])
        t2 = jnp.maximum(_dot_t(g, wi1_ref[...]) + bi1_ref[...], 0.0)
        issues_ref[...] = _sigmoid(_dot_t(t2, wi2_ref[...]) + bi2_ref[...])


def _tc_final(za, zb, u, dinv, b, ws1, bs1, ws2, bs2, wi1, bi1, wi2, bi2):
    full = lambda shp: pl.BlockSpec(shp, lambda i: (0, 0))
    return pl.pallas_call(
        _tc_final_body,
        grid=(_GRID,),
        in_specs=[
            pl.BlockSpec((_BLK, D), lambda i: (i, 0)),
            pl.BlockSpec((_BLK, D), lambda i: (i, 0)),
            pl.BlockSpec((_BLK, D), lambda i: (i, 0)),
            pl.BlockSpec((_BLK, 1), lambda i: (i, 0)),
            full((1, D)),
            full((D // 2, D)), full((1, D // 2)),
            full((1, D // 2)), full((1, 1)),
            full((D // 2, D)), full((1, D // 2)),
            full((10, D // 2)), full((1, 10)),
        ],
        out_specs=[full((1, 1)), full((1, 10))],
        out_shape=[
            jax.ShapeDtypeStruct((1, 1), _f32),
            jax.ShapeDtypeStruct((1, 10), _f32),
        ],
        scratch_shapes=[pltpu.VMEM((1, D), _f32)],
    )(za, zb, u, dinv, b, ws1, bs1, ws2, bs2, wi1, bi1, wi2, bi2)


def kernel(x, edge_index, W1, b1, W2, b2, W3, b3,
           Ws1, bs1, Ws2, bs2, Wi1, bi1, Wi2, bi2):
    xp = jnp.pad(x, ((0, NP - N), (0, 0)))
    src = edge_index[0]
    dst = edge_index[1]
    b1r = b1.reshape(1, D)
    b2r = b2.reshape(1, D)
    b3r = b3.reshape(1, D)
    bs1r = bs1.reshape(1, D // 2)
    bs2r = bs2.reshape(1, 1)
    bi1r = bi1.reshape(1, D // 2)
    bi2r = bi2.reshape(1, 10)

    dega, degb = _deg_kernel(dst)
    u1, dinv = _tc1(xp, dega.reshape(NP, 1), degb.reshape(NP, 1), W1)
    za, zb = _edge_kernel(u1, src, dst)
    u2 = _tc_mid(za, zb, u1, dinv, b1r, W2)
    za, zb = _edge_kernel(u2, src, dst)
    u3 = _tc_mid(za, zb, u2, dinv, b2r, W3)
    za, zb = _edge_kernel(u3, src, dst)
    score, issues = _tc_final(za, zb, u3, dinv, b3r,
                              Ws1, bs1r, Ws2, bs2r, Wi1, bi1r, Wi2, bi2r)
    return (score, issues)


# speedup vs baseline: 24.9877x; 1.7907x over previous
import functools

import jax
import jax.numpy as jnp
from jax import lax
from jax.experimental import pallas as pl
from jax.experimental.pallas import tpu as pltpu
from jax.experimental.pallas import tpu_sc as plsc

N = 10000
NP = 10240
D = 128
E = 320000
EB = 125
NC, NS = 2, 16
CPT = 80
NBUF = 2
NGRP = CPT // NBUF
RPS = NP // NS
WCH = RPS // 128

_mesh = plsc.VectorSubcoreMesh(
    core_axis_name="c", subcore_axis_name="s", num_cores=NC, num_subcores=NS)

_f32 = jnp.float32


@functools.partial(
    pl.kernel,
    out_type=(jax.ShapeDtypeStruct((NP,), _f32),
              jax.ShapeDtypeStruct((NP,), _f32)),
    mesh=_mesh,
    scratch_types=[
        pltpu.VMEM((CPT, EB), jnp.int32),
        pltpu.VMEM((128,), _f32),
        pltpu.VMEM((RPS,), _f32),
        pltpu.VMEM_SHARED((NP,), _f32),
        pltpu.SemaphoreType.DMA,
    ],
)
def _deg_kernel(dst_hbm, out_a, out_b, dst_slab, ones_v, stage_v, acc_sh, sem):
    c = lax.axis_index("c")
    s = lax.axis_index("s")
    base = (c * NS + s) * CPT

    pltpu.sync_copy(dst_hbm.at[pl.ds(base, CPT)], dst_slab)

    def _fill(t, _):
        ones_v[pl.ds(t * 16, 16)] = jnp.full((16,), 1.0, _f32)
        return 0
    lax.fori_loop(0, 8, _fill, 0)

    def _zero(t, _):
        stage_v[pl.ds(t * 16, 16)] = jnp.zeros((16,), _f32)
        return 0
    lax.fori_loop(0, RPS // 16, _zero, 0)

    pltpu.sync_copy(stage_v, acc_sh.at[pl.ds(s * RPS, RPS)])
    plsc.subcore_barrier()

    def _grp(g, _):
        for b in range(8):
            j = g * 8 + b
            pltpu.async_copy(ones_v.at[pl.ds(0, EB)],
                             acc_sh.at[dst_slab.at[j]], sem, add=True)
        for b in range(8):
            j = g * 8 + b
            pltpu.make_async_copy(ones_v.at[pl.ds(0, EB)],
                                  acc_sh.at[dst_slab.at[j]], sem).wait()
        return 0
    lax.fori_loop(0, CPT // 8, _grp, 0)
    plsc.subcore_barrier()

    pltpu.sync_copy(acc_sh.at[pl.ds(s * RPS, RPS)], stage_v)

    @pl.when(c == 0)
    def _():
        pltpu.sync_copy(stage_v, out_a.at[pl.ds(s * RPS, RPS)])

    @pl.when(c == 1)
    def _():
        pltpu.sync_copy(stage_v, out_b.at[pl.ds(s * RPS, RPS)])


@functools.partial(
    pl.kernel,
    out_type=(jax.ShapeDtypeStruct((NP, D), _f32),
              jax.ShapeDtypeStruct((NP, D), _f32)),
    mesh=_mesh,
    scratch_types=[
        pltpu.VMEM((CPT, EB), jnp.int32),
        pltpu.VMEM((CPT // 2, EB), jnp.int32),
        pltpu.VMEM((2 * EB, D), _f32),
        pltpu.VMEM_SHARED((NP, D), _f32),
        pltpu.SemaphoreType.DMA,
    ],
)
def _edge_kernel(u_hbm, src_hbm, dst_hbm, out_a, out_b,
                 src_slab, dst_slab, rb, acc_sh, semg):
    c = lax.axis_index("c")
    s = lax.axis_index("s")
    base = (c * NS + s) * CPT

    pltpu.sync_copy(src_hbm.at[pl.ds(base, CPT)], src_slab)
    pltpu.sync_copy(dst_hbm.at[pl.ds(base, CPT // 2)], dst_slab)

    def _zero(t, _):
        rb[t // 8, pl.ds((t % 8) * 16, 16)] = jnp.zeros((16,), _f32)
        return 0
    lax.fori_loop(0, 128 * D // 16, _zero, 0)

    def _zacc(k, _):
        pltpu.sync_copy(rb.at[pl.ds(0, 128)],
                        acc_sh.at[pl.ds(s * RPS + k * 128, 128)])
        return 0
    lax.fori_loop(0, WCH, _zacc, 0)
    plsc.subcore_barrier()

    def _g_start(b, j):
        pltpu.async_copy(u_hbm.at[src_slab.at[j]],
                         rb.at[pl.ds(b * EB, EB)], semg)

    def _g_wait(b, j):
        pltpu.make_async_copy(u_hbm.at[src_slab.at[j]],
                              rb.at[pl.ds(b * EB, EB)], semg).wait()

    def _scatter(b, j):
        pltpu.sync_copy(rb.at[pl.ds(b * EB, EB)],
                        acc_sh.at[dst_slab.at[j % (CPT // 2)]], add=True)

    _g_start(0, 0)

    def _pair(g, _):
        @pl.when(g == CPT // 4)
        def _():
            pltpu.sync_copy(dst_hbm.at[pl.ds(base + CPT // 2, CPT // 2)],
                            dst_slab)

        for b in range(2):
            j = 2 * g + b
            _g_wait(b, j)

            @pl.when(j < CPT - 1)
            def _():
                _g_start(1 - b, j + 1)
            _scatter(b, j)
        return 0
    lax.fori_loop(0, CPT // 2, _pair, 0)
    plsc.subcore_barrier()

    def _wout(k, _):
        pltpu.sync_copy(acc_sh.at[pl.ds(s * RPS + k * 128, 128)],
                        rb.at[pl.ds(0, 128)])

        @pl.when(c == 0)
        def _():
            pltpu.sync_copy(rb.at[pl.ds(0, 128)],
                            out_a.at[pl.ds(s * RPS + k * 128, 128)])

        @pl.when(c == 1)
        def _():
            pltpu.sync_copy(rb.at[pl.ds(0, 128)],
                            out_b.at[pl.ds(s * RPS + k * 128, 128)])
        return 0
    lax.fori_loop(0, WCH, _wout, 0)


_BLK = 1024
_GRID = NP // _BLK


def _dot_t(a, w):
    return lax.dot_general(a, w, (((1,), (1,)), ((), ())),
                           preferred_element_type=_f32)


def _tc1_body(x_ref, da_ref, db_ref, w_ref, u_ref, dinv_ref):
    dinv = lax.rsqrt(da_ref[...] + db_ref[...] + 1.0)
    u_ref[...] = dinv * _dot_t(x_ref[...], w_ref[...])
    dinv_ref[...] = dinv


def _tc1(xp, dega, degb, w1):
    return pl.pallas_call(
        _tc1_body,
        grid=(_GRID,),
        in_specs=[
            pl.BlockSpec((_BLK, D), lambda i: (i, 0)),
            pl.BlockSpec((_BLK, 1), lambda i: (i, 0)),
            pl.BlockSpec((_BLK, 1), lambda i: (i, 0)),
            pl.BlockSpec((D, D), lambda i: (0, 0)),
        ],
        out_specs=[
            pl.BlockSpec((_BLK, D), lambda i: (i, 0)),
            pl.BlockSpec((_BLK, 1), lambda i: (i, 0)),
        ],
        out_shape=[
            jax.ShapeDtypeStruct((NP, D), _f32),
            jax.ShapeDtypeStruct((NP, 1), _f32),
        ],
    )(xp, dega, degb, w1)


def _tc_mid_body(za_ref, zb_ref, u_ref, dinv_ref, b_ref, w_ref, un_ref):
    dinv = dinv_ref[...]
    h = dinv * (za_ref[...] + zb_ref[...] + u_ref[...]) + b_ref[...]
    h = jnp.maximum(h, 0.0)
    un_ref[...] = dinv * _dot_t(h, w_ref[...])


def _tc_mid(za, zb, u, dinv, b, w):
    return pl.pallas_call(
        _tc_mid_body,
        grid=(_GRID,),
        in_specs=[
            pl.BlockSpec((_BLK, D), lambda i: (i, 0)),
            pl.BlockSpec((_BLK, D), lambda i: (i, 0)),
            pl.BlockSpec((_BLK, D), lambda i: (i, 0)),
            pl.BlockSpec((_BLK, 1), lambda i: (i, 0)),
            pl.BlockSpec((1, D), lambda i: (0, 0)),
            pl.BlockSpec((D, D), lambda i: (0, 0)),
        ],
        out_specs=pl.BlockSpec((_BLK, D), lambda i: (i, 0)),
        out_shape=jax.ShapeDtypeStruct((NP, D), _f32),
    )(za, zb, u, dinv, b, w)


def _sigmoid(t):
    return 1.0 / (1.0 + jnp.exp(-t))


def _tc_final_body(za_ref, zb_ref, u_ref, dinv_ref, b_ref,
                   ws1_ref, bs1_ref, ws2_ref, bs2_ref,
                   wi1_ref, bi1_ref, wi2_ref, bi2_ref,
                   score_ref, issues_ref, acc_ref):
    i = pl.program_id(0)
    h = dinv_ref[...] * (za_ref[...] + zb_ref[...] + u_ref[...]) + b_ref[...]
    h = jnp.maximum(h, 0.0)
    rows = lax.broadcasted_iota(jnp.int32, (_BLK, 1), 0) + i * _BLK
    h = h * (rows < N).astype(_f32)

    @pl.when(i == 0)
    def _():
        acc_ref[...] = jnp.zeros((1, D), _f32)

    acc_ref[...] += jnp.sum(h, axis=0, keepdims=True)

    @pl.when(i == _GRID - 1)
    def _():
        g = acc_ref[...] * (1.0 / N)
        t1 = jnp.maximum(_dot_t(g, ws1_ref[...]) + bs1_ref[...], 0.0)
        s_lin = jnp.sum(t1 * ws2_ref[...], axis=1, keepdims=True)
        score_ref[...] = _sigmoid(s_lin + bs2_ref[...])
        t2 = jnp.maximum(_dot_t(g, wi1_ref[...]) + bi1_ref[...], 0.0)
        issues_ref[...] = _sigmoid(_dot_t(t2, wi2_ref[...]) + bi2_ref[...])


def _tc_final(za, zb, u, dinv, b, ws1, bs1, ws2, bs2, wi1, bi1, wi2, bi2):
    full = lambda shp: pl.BlockSpec(shp, lambda i: (0, 0))
    return pl.pallas_call(
        _tc_final_body,
        grid=(_GRID,),
        in_specs=[
            pl.BlockSpec((_BLK, D), lambda i: (i, 0)),
            pl.BlockSpec((_BLK, D), lambda i: (i, 0)),
            pl.BlockSpec((_BLK, D), lambda i: (i, 0)),
            pl.BlockSpec((_BLK, 1), lambda i: (i, 0)),
            full((1, D)),
            full((D // 2, D)), full((1, D // 2)),
            full((1, D // 2)), full((1, 1)),
            full((D // 2, D)), full((1, D // 2)),
            full((10, D // 2)), full((1, 10)),
        ],
        out_specs=[full((1, 1)), full((1, 10))],
        out_shape=[
            jax.ShapeDtypeStruct((1, 1), _f32),
            jax.ShapeDtypeStruct((1, 10), _f32),
        ],
        scratch_shapes=[pltpu.VMEM((1, D), _f32)],
    )(za, zb, u, dinv, b, ws1, bs1, ws2, bs2, wi1, bi1, wi2, bi2)


def kernel(x, edge_index, W1, b1, W2, b2, W3, b3,
           Ws1, bs1, Ws2, bs2, Wi1, bi1, Wi2, bi2):
    xp = jnp.pad(x, ((0, NP - N), (0, 0)))
    src = edge_index[0].reshape(E // EB, EB)
    dst = edge_index[1].reshape(E // EB, EB)
    b1r = b1.reshape(1, D)
    b2r = b2.reshape(1, D)
    b3r = b3.reshape(1, D)
    bs1r = bs1.reshape(1, D // 2)
    bs2r = bs2.reshape(1, 1)
    bi1r = bi1.reshape(1, D // 2)
    bi2r = bi2.reshape(1, 10)

    dega, degb = _deg_kernel(dst)
    u1, dinv = _tc1(xp, dega.reshape(NP, 1), degb.reshape(NP, 1), W1)
    za, zb = _edge_kernel(u1, src, dst)
    u2 = _tc_mid(za, zb, u1, dinv, b1r, W2)
    za, zb = _edge_kernel(u2, src, dst)
    u3 = _tc_mid(za, zb, u2, dinv, b2r, W3)
    za, zb = _edge_kernel(u3, src, dst)
    score, issues = _tc_final(za, zb, u3, dinv, b3r,
                              Ws1, bs1r, Ws2, bs2r, Wi1, bi1r, Wi2, bi2r)
    return (score, issues)
